# initial kernel scaffold (unmeasured)
import jax
import jax.numpy as jnp
from jax import lax
from jax.experimental import pallas as pl
from jax.experimental.pallas import tpu as pltpu

N_DEV = 8
M_PER = 512
K = 4096
N_PER = 1024


def kernel(x, w_mat, scale_x, scale_w):
    scale = (scale_x[0] * scale_w[0]).reshape(1, 1).astype(jnp.float32)
    x_bf = x.astype(jnp.bfloat16)
    w_bf = w_mat.astype(jnp.bfloat16)

    def body(x_ref, w_hbm, scale_ref, out_ref,
             wt_ref, send_ref, recv_ref,
             load_sem, send_sems, recv_sems):
        me = lax.axis_index("i")

        barrier = pltpu.get_barrier_semaphore()
        for d in range(1, N_DEV):
            pl.semaphore_signal(
                barrier, inc=1,
                device_id=((me + d) % N_DEV,),
                device_id_type=pl.DeviceIdType.MESH,
            )
        pl.semaphore_wait(barrier, N_DEV - 1)

        sc = scale_ref[0, 0]

        for step in range(N_DEV):
            j = (me + step) % N_DEV
            cp = pltpu.make_async_copy(
                w_hbm.at[:, pl.ds(j * N_PER, N_PER)],
                wt_ref.at[step % 2],
                load_sem,
            )
            cp.start()
            cp.wait()
            acc = jnp.dot(x_ref[...], wt_ref[step % 2],
                          preferred_element_type=jnp.float32)
            z = acc * sc
            y = z * jax.nn.sigmoid(z)
            if step == 0:
                out_ref[pl.ds(me * M_PER, M_PER), :] = y
            else:
                send_ref[step] = y.astype(jnp.bfloat16)
                rdma = pltpu.make_async_remote_copy(
                    src_ref=send_ref.at[step],
                    dst_ref=recv_ref.at[me],
                    send_sem=send_sems.at[step],
                    recv_sem=recv_sems.at[me],
                    device_id=(j,),
                    device_id_type=pl.DeviceIdType.MESH,
                )
                rdma.start()
                rdma.wait_send()

        for d in range(1, N_DEV):
            s = (me + d) % N_DEV
            recv = pltpu.make_async_remote_copy(
                src_ref=send_ref.at[0],
                dst_ref=recv_ref.at[s],
                send_sem=send_sems.at[0],
                recv_sem=recv_sems.at[s],
                device_id=(me,),
                device_id_type=pl.DeviceIdType.MESH,
            )
            recv.wait_recv()
            out_ref[pl.ds(s * M_PER, M_PER), :] = recv_ref[s].astype(jnp.float32)

    return pl.pallas_call(
        body,
        out_shape=jax.ShapeDtypeStruct((N_DEV * M_PER, N_PER), jnp.float32),
        in_specs=[
            pl.BlockSpec(memory_space=pltpu.VMEM),
            pl.BlockSpec(memory_space=pltpu.ANY),
            pl.BlockSpec(memory_space=pltpu.SMEM),
        ],
        out_specs=pl.BlockSpec(memory_space=pltpu.VMEM),
        scratch_shapes=[
            pltpu.VMEM((2, K, N_PER), jnp.bfloat16),
            pltpu.VMEM((N_DEV, M_PER, N_PER), jnp.bfloat16),
            pltpu.VMEM((N_DEV, M_PER, N_PER), jnp.bfloat16),
            pltpu.SemaphoreType.DMA,
            pltpu.SemaphoreType.DMA((N_DEV,)),
            pltpu.SemaphoreType.DMA((N_DEV,)),
        ],
        compiler_params=pltpu.CompilerParams(collective_id=0),
    )(x_bf, w_bf, scale)


# baseline (device time: 247997 ns/iter reference)
import jax
import jax.numpy as jnp
from jax import lax
from jax.experimental import pallas as pl
from jax.experimental.pallas import tpu as pltpu

N_DEV = 8
M_PER = 512
K = 4096
N_PER = 1024


def kernel(x, w_mat, scale_x, scale_w):
    scale = (scale_x[0] * scale_w[0]).reshape(1, 1).astype(jnp.float32)
    x_bf = x.astype(jnp.bfloat16)
    w_bf = w_mat.astype(jnp.bfloat16)

    def body(x_ref, w_hbm, scale_ref, out_ref,
             wt_ref, send_ref, recv_ref,
             load_sem, send_sems, recv_sems):
        me = lax.axis_index("i")

        barrier = pltpu.get_barrier_semaphore()
        for d in range(1, N_DEV):
            pl.semaphore_signal(
                barrier, inc=1,
                device_id=((me + d) % N_DEV,),
                device_id_type=pl.DeviceIdType.MESH,
            )
        pl.semaphore_wait(barrier, N_DEV - 1)

        sc = scale_ref[0, 0]

        for step in range(N_DEV):
            j = (me + step) % N_DEV
            cp = pltpu.make_async_copy(
                w_hbm.at[:, pl.ds(j * N_PER, N_PER)],
                wt_ref.at[step % 2],
                load_sem,
            )
            cp.start()
            cp.wait()
            acc = jnp.dot(x_ref[...], wt_ref[step % 2],
                          preferred_element_type=jnp.float32)
            z = acc * sc
            y = z * jax.nn.sigmoid(z)
            if step == 0:
                out_ref[pl.ds(me * M_PER, M_PER), :] = y
            else:
                send_ref[step] = y.astype(jnp.bfloat16)
                rdma = pltpu.make_async_remote_copy(
                    src_ref=send_ref.at[step],
                    dst_ref=recv_ref.at[me],
                    send_sem=send_sems.at[step],
                    recv_sem=recv_sems.at[me],
                    device_id=(j,),
                    device_id_type=pl.DeviceIdType.MESH,
                )
                rdma.start()
                rdma.wait_send()

        for d in range(1, N_DEV):
            s = (me + d) % N_DEV
            recv = pltpu.make_async_remote_copy(
                src_ref=send_ref.at[0],
                dst_ref=recv_ref.at[s],
                send_sem=send_sems.at[0],
                recv_sem=recv_sems.at[s],
                device_id=(me,),
                device_id_type=pl.DeviceIdType.MESH,
            )
            recv.wait_recv()
            out_ref[pl.ds(s * M_PER, M_PER), :] = recv_ref[s].astype(jnp.float32)

    return pl.pallas_call(
        body,
        out_shape=jax.ShapeDtypeStruct((N_DEV * M_PER, N_PER), jnp.float32),
        in_specs=[
            pl.BlockSpec(memory_space=pltpu.VMEM),
            pl.BlockSpec(memory_space=pl.ANY),
            pl.BlockSpec(memory_space=pltpu.SMEM),
        ],
        out_specs=pl.BlockSpec(memory_space=pltpu.VMEM),
        scratch_shapes=[
            pltpu.VMEM((2, K, N_PER), jnp.bfloat16),
            pltpu.VMEM((N_DEV, M_PER, N_PER), jnp.bfloat16),
            pltpu.VMEM((N_DEV, M_PER, N_PER), jnp.bfloat16),
            pltpu.SemaphoreType.DMA,
            pltpu.SemaphoreType.DMA((N_DEV,)),
            pltpu.SemaphoreType.DMA((N_DEV,)),
        ],
        compiler_params=pltpu.CompilerParams(
            collective_id=0,
            vmem_limit_bytes=60 * 1024 * 1024,
        ),
    )(x_bf, w_bf, scale)
